# baseline (device time: 8924 ns/iter reference)
import jax
import jax.numpy as jnp
from jax import lax
from jax.experimental import pallas as pl
from jax.experimental.pallas import tpu as pltpu

N_GLOBAL = 1024
EPS = 1e-5
NCHUNK = 4


def kernel(x, gamma, beta):
    m, n = x.shape
    cm = m // NCHUNK
    r = cm // 128

    def body(x_ref, g_ref, b_ref, out_ref, stats_ref, recv_ref, send_sems, recv_sems):
        my_x = lax.axis_index("x")
        my_y = lax.axis_index("y")
        nbr = (my_x, 1 - my_y)

        barrier_sem = pltpu.get_barrier_semaphore()
        pl.semaphore_signal(
            barrier_sem, inc=1, device_id=nbr, device_id_type=pl.DeviceIdType.MESH
        )
        pl.semaphore_wait(barrier_sem, 1)

        rdmas = []
        for c in range(NCHUNK):
            lo, hi = c * cm, (c + 1) * cm
            xa = x_ref[lo:hi, :]
            s = jnp.sum(xa, axis=1)
            q = jnp.sum(xa * xa, axis=1)
            base = c * 2 * r
            stats_ref[base : base + r, :] = s.reshape(r, 128)
            stats_ref[base + r : base + 2 * r, :] = q.reshape(r, 128)
            rdma = pltpu.make_async_remote_copy(
                src_ref=stats_ref.at[base : base + 2 * r],
                dst_ref=recv_ref.at[base : base + 2 * r],
                send_sem=send_sems.at[c],
                recv_sem=recv_sems.at[c],
                device_id=nbr,
                device_id_type=pl.DeviceIdType.MESH,
            )
            rdma.start()
            rdmas.append(rdma)

        lane = lax.broadcasted_iota(jnp.int32, (128, 128), 1)
        sub = lax.broadcasted_iota(jnp.int32, (128, 128), 0)
        diag = lane == sub
        for c in range(NCHUNK):
            rdmas[c].wait()
            base = c * 2 * r
            tot = stats_ref[base : base + 2 * r, :] + recv_ref[base : base + 2 * r, :]
            for a in range(r):
                srow = tot[a : a + 1, :]
                qrow = tot[r + a : r + a + 1, :]
                scol = jnp.sum(
                    jnp.where(diag, jnp.broadcast_to(srow, (128, 128)), 0.0),
                    axis=1,
                    keepdims=True,
                )
                qcol = jnp.sum(
                    jnp.where(diag, jnp.broadcast_to(qrow, (128, 128)), 0.0),
                    axis=1,
                    keepdims=True,
                )
                mean = scol / N_GLOBAL
                var = qcol / N_GLOBAL - mean * mean
                inv = lax.rsqrt(var + EPS)
                lo, hi = c * cm + a * 128, c * cm + (a + 1) * 128
                xa = x_ref[lo:hi, :]
                out_ref[lo:hi, :] = g_ref[0:1, :] * ((xa - mean) * inv) + b_ref[0:1, :]

    return pl.pallas_call(
        body,
        out_shape=jax.ShapeDtypeStruct((m, n), x.dtype),
        in_specs=[
            pl.BlockSpec(memory_space=pltpu.VMEM),
            pl.BlockSpec(memory_space=pltpu.VMEM),
            pl.BlockSpec(memory_space=pltpu.VMEM),
        ],
        out_specs=pl.BlockSpec(memory_space=pltpu.VMEM),
        scratch_shapes=[
            pltpu.VMEM((16, 128), jnp.float32),
            pltpu.VMEM((16, 128), jnp.float32),
            pltpu.SemaphoreType.DMA((NCHUNK,)),
            pltpu.SemaphoreType.DMA((NCHUNK,)),
        ],
        compiler_params=pltpu.CompilerParams(collective_id=0),
    )(x, gamma.reshape(1, n), beta.reshape(1, n))


# device time: 8800 ns/iter; 1.0141x vs baseline; 1.0141x over previous
import jax
import jax.numpy as jnp
from jax import lax
from jax.experimental import pallas as pl
from jax.experimental.pallas import tpu as pltpu

N_GLOBAL = 1024
EPS = 1e-5


def kernel(x, gamma, beta):
    m, n = x.shape

    def body(x_ref, g_ref, b_ref, out_ref, stats_ref, recv_ref, send_sem, recv_sem):
        my_x = lax.axis_index("x")
        my_y = lax.axis_index("y")
        nbr = (my_x, 1 - my_y)

        barrier_sem = pltpu.get_barrier_semaphore()
        pl.semaphore_signal(
            barrier_sem, inc=1, device_id=nbr, device_id_type=pl.DeviceIdType.MESH
        )
        pl.semaphore_wait(barrier_sem, 1)

        xv = x_ref[:, :]
        s = jnp.sum(xv, axis=1)
        q = jnp.sum(xv * xv, axis=1)
        stats_ref[0:8, :] = s.reshape(8, 128)
        stats_ref[8:16, :] = q.reshape(8, 128)

        rdma = pltpu.make_async_remote_copy(
            src_ref=stats_ref,
            dst_ref=recv_ref,
            send_sem=send_sem,
            recv_sem=recv_sem,
            device_id=nbr,
            device_id_type=pl.DeviceIdType.MESH,
        )
        rdma.start()
        rdma.wait()

        tot = stats_ref[:, :] + recv_ref[:, :]
        meanl = tot[0:8, :] / N_GLOBAL
        invl = lax.rsqrt(tot[8:16, :] / N_GLOBAL - meanl * meanl + EPS)
        ab = jnp.concatenate([invl, -meanl * invl], axis=0)

        lane = lax.broadcasted_iota(jnp.int32, (128, 128), 1)
        sub = lax.broadcasted_iota(jnp.int32, (128, 128), 0)
        eye = (lane == sub).astype(jnp.float32)
        tt = lax.dot_general(
            eye,
            ab,
            (((1,), (1,)), ((), ())),
            preferred_element_type=jnp.float32,
        )

        gv = g_ref[0:1, :]
        bv = b_ref[0:1, :]
        for a in range(8):
            inv_col = tt[:, a : a + 1]
            off_col = tt[:, 8 + a : 9 + a]
            lo, hi = a * 128, (a + 1) * 128
            xa = xv[lo:hi, :]
            out_ref[lo:hi, :] = gv * (xa * inv_col + off_col) + bv

    return pl.pallas_call(
        body,
        out_shape=jax.ShapeDtypeStruct((m, n), x.dtype),
        in_specs=[
            pl.BlockSpec(memory_space=pltpu.VMEM),
            pl.BlockSpec(memory_space=pltpu.VMEM),
            pl.BlockSpec(memory_space=pltpu.VMEM),
        ],
        out_specs=pl.BlockSpec(memory_space=pltpu.VMEM),
        scratch_shapes=[
            pltpu.VMEM((16, 128), jnp.float32),
            pltpu.VMEM((16, 128), jnp.float32),
            pltpu.SemaphoreType.DMA,
            pltpu.SemaphoreType.DMA,
        ],
        compiler_params=pltpu.CompilerParams(collective_id=0),
    )(x, gamma.reshape(1, n), beta.reshape(1, n))
